# baseline (speedup 1.0000x reference)
import functools

import jax
import jax.numpy as jnp
from jax import lax
from jax.experimental import pallas as pl
from jax.experimental.pallas import tpu as pltpu, tpu_sc as plsc

VOCAB = 100000
EMBED_DIM = 64
BATCH = 4096
CTX = 50
XW = CTX + 1

NC, NS, LANES = 2, 16, 16
NW = NC * NS
B_PER_W = BATCH // NW
CB = 8
N_CHUNKS = B_PER_W // CB
ROWS_PER_CHUNK = CB * CTX
ROWS_PER_W = B_PER_W * CTX
IDX_CHUNK = 128
DIV50_MUL, DIV50_SHIFT = 20972, 20

CPAD = ROWS_PER_CHUNK + LANES
OW = B_PER_W + 1


def _body(x_hbm, wtab_hbm, ctab_hbm, out_hbm,
          xs_v, widx_v, cidx_v, wrows_v, crows_v, out_v, gsem):
    wid = lax.axis_index("s") * NC + lax.axis_index("c")
    lane_iota = lax.iota(jnp.int32, LANES)
    row0 = wid * B_PER_W

    pltpu.sync_copy(x_hbm.at[:, pl.ds(row0, B_PER_W)], xs_v)
    zvec = jnp.zeros((LANES,), jnp.int32)
    for k in range(B_PER_W // LANES):
        widx_v[pl.ds(k * LANES, LANES)] = plsc.load_gather(
            xs_v, [zvec, k * LANES + lane_iota])

    def extract_body(g2, _):
        o = g2 * LANES + lane_iota
        j = (o * DIV50_MUL) >> DIV50_SHIFT
        p1 = o - j * CTX + 1
        cidx_v[pl.ds(g2 * LANES, LANES)] = plsc.load_gather(xs_v, [p1, j])
        return ()

    lax.fori_loop(0, ROWS_PER_W // LANES, extract_body, (), unroll=False)

    def fire_gathers(k):
        p = lax.rem(k, 2)
        pltpu.async_copy(
            wtab_hbm.at[widx_v.at[pl.ds(k * CB, CB)]],
            wrows_v.at[pl.ds(p * CB, CB)], gsem)
        off = 0
        while off < ROWS_PER_CHUNK:
            n = min(IDX_CHUNK, ROWS_PER_CHUNK - off)
            pltpu.async_copy(
                ctab_hbm.at[cidx_v.at[pl.ds(k * ROWS_PER_CHUNK + off, n)]],
                crows_v.at[pl.ds(p * CPAD + off, n)], gsem)
            off += n

    def drain_gathers():
        pltpu.make_async_copy(ctab_hbm.at[pl.ds(0, ROWS_PER_CHUNK)],
                              crows_v.at[pl.ds(0, ROWS_PER_CHUNK)],
                              gsem).wait()
        pltpu.make_async_copy(wtab_hbm.at[pl.ds(0, CB)],
                              wrows_v.at[pl.ds(0, CB)], gsem).wait()

    fire_gathers(jnp.int32(0))

    def chunk_body(k, _):
        p = lax.rem(k, 2)

        @pl.when(k + 1 < N_CHUNKS)
        def _():
            fire_gathers(k + 1)

        drain_gathers()

        poff_c = p * CPAD
        poff_w = p * CB

        def batch_body(b, _):
            bvec = jnp.broadcast_to(poff_w + b, (LANES,)).astype(jnp.int32)
            bw = jnp.broadcast_to(k * CB + b, (LANES,)).astype(jnp.int32)
            rows = [poff_c + b * CTX + g * LANES + lane_iota for g in range(4)]
            accs = [jnp.zeros((LANES,), jnp.float32) for _ in range(4)]
            for d in range(EMBED_DIM):
                col = jnp.where(lane_iota >= EMBED_DIM - d,
                                lane_iota + (d - EMBED_DIM),
                                lane_iota + d)
                w_d = plsc.load_gather(wrows_v, [bvec, col])
                for g in range(4):
                    v = plsc.load_gather(crows_v, [rows[g], col])
                    accs[g] = accs[g] + v * w_d
            for g in range(4):
                sig = 1.0 / (1.0 + jnp.exp(-accs[g]))
                cvec = g * LANES + lane_iota
                if g * LANES + LANES <= CTX:
                    plsc.store_scatter(out_v, [cvec, bw], sig)
                else:
                    plsc.store_scatter(out_v, [cvec, bw], sig,
                                       mask=lane_iota < (CTX - g * LANES))
            return ()

        lax.fori_loop(0, CB, batch_body, (), unroll=2)
        return ()

    lax.fori_loop(0, N_CHUNKS, chunk_body, (), unroll=False)
    pltpu.sync_copy(out_v.at[:, pl.ds(0, B_PER_W)],
                    out_hbm.at[:, pl.ds(row0, B_PER_W)])


@functools.partial(
    pl.kernel,
    out_type=jax.ShapeDtypeStruct((CTX, BATCH), jnp.float32),
    mesh=plsc.VectorSubcoreMesh(core_axis_name="c", subcore_axis_name="s"),
    compiler_params=pltpu.CompilerParams(
        needs_layout_passes=False, use_tc_tiling_on_sc=False),
    scratch_types=[
        pltpu.VMEM((XW, B_PER_W), jnp.int32),
        pltpu.VMEM((B_PER_W,), jnp.int32),
        pltpu.VMEM((ROWS_PER_W,), jnp.int32),
        pltpu.VMEM((2 * CB, EMBED_DIM), jnp.float32),
        pltpu.VMEM((2 * CPAD, EMBED_DIM), jnp.float32),
        pltpu.VMEM((CTX, OW), jnp.float32),
        pltpu.SemaphoreType.DMA,
    ],
)
def _w2v_sc(x_hbm, wtab_hbm, ctab_hbm, out_hbm,
            xs_v, widx_v, cidx_v, wrows_v, crows_v, out_v, gsem):
    _body(x_hbm, wtab_hbm, ctab_hbm, out_hbm,
          xs_v, widx_v, cidx_v, wrows_v, crows_v, out_v, gsem)


def kernel(x, word_emb, ctx_emb):
    out = _w2v_sc(x.T, word_emb, ctx_emb)
    return out.T[:, :, None]


# speedup vs baseline: 1.0018x; 1.0018x over previous
import functools

import jax
import jax.numpy as jnp
from jax import lax
from jax.experimental import pallas as pl
from jax.experimental.pallas import tpu as pltpu, tpu_sc as plsc

VOCAB = 100000
EMBED_DIM = 64
BATCH = 4096
CTX = 50
XW = CTX + 1

NC, NS, LANES = 2, 16, 16
NW = NC * NS
B_PER_W = BATCH // NW
CB = 8
N_CHUNKS = B_PER_W // CB
ROWS_PER_CHUNK = CB * CTX
ROWS_PER_W = B_PER_W * CTX
IDX_CHUNK = 128
DIV50_MUL, DIV50_SHIFT = 20972, 20

CPAD = ROWS_PER_CHUNK + LANES
OW = B_PER_W + 1


def _body(x_hbm, wtab_hbm, ctab_hbm, out_hbm,
          xs_v, widx_v, cidx_v, wrows_v, crows_v, out_v, gsem):
    wid = lax.axis_index("s") * NC + lax.axis_index("c")
    lane_iota = lax.iota(jnp.int32, LANES)
    row0 = wid * B_PER_W

    pltpu.sync_copy(x_hbm.at[:, pl.ds(row0, B_PER_W)], xs_v)
    zvec = jnp.zeros((LANES,), jnp.int32)
    for k in range(B_PER_W // LANES):
        widx_v[pl.ds(k * LANES, LANES)] = plsc.load_gather(
            xs_v, [zvec, k * LANES + lane_iota])

    def extract_body(g2, _):
        o = g2 * LANES + lane_iota
        j = (o * DIV50_MUL) >> DIV50_SHIFT
        p1 = o - j * CTX + 1
        cidx_v[pl.ds(g2 * LANES, LANES)] = plsc.load_gather(xs_v, [p1, j])
        return ()

    lax.fori_loop(0, ROWS_PER_W // LANES, extract_body, (), unroll=False)

    def fire_gathers(k):
        p = lax.rem(k, 2)
        pltpu.async_copy(
            wtab_hbm.at[widx_v.at[pl.ds(k * CB, CB)]],
            wrows_v.at[pl.ds(p * CB, CB)], gsem)
        off = 0
        while off < ROWS_PER_CHUNK:
            n = min(IDX_CHUNK, ROWS_PER_CHUNK - off)
            pltpu.async_copy(
                ctab_hbm.at[cidx_v.at[pl.ds(k * ROWS_PER_CHUNK + off, n)]],
                crows_v.at[pl.ds(p * CPAD + off, n)], gsem)
            off += n

    def drain_gathers():
        pltpu.make_async_copy(ctab_hbm.at[pl.ds(0, ROWS_PER_CHUNK)],
                              crows_v.at[pl.ds(0, ROWS_PER_CHUNK)],
                              gsem).wait()
        pltpu.make_async_copy(wtab_hbm.at[pl.ds(0, CB)],
                              wrows_v.at[pl.ds(0, CB)], gsem).wait()

    fire_gathers(jnp.int32(0))

    def chunk_body(k, _):
        p = lax.rem(k, 2)

        @pl.when(k + 1 < N_CHUNKS)
        def _():
            fire_gathers(k + 1)

        drain_gathers()

        poff_c = p * CPAD
        poff_w = p * CB

        def batch_body(b, _):
            bvec = jnp.broadcast_to(poff_w + b, (LANES,)).astype(jnp.int32)
            bw = jnp.broadcast_to(k * CB + b, (LANES,)).astype(jnp.int32)
            rows = [poff_c + b * CTX + g * LANES + lane_iota for g in range(4)]
            accs = [jnp.zeros((LANES,), jnp.float32) for _ in range(4)]
            for d in range(EMBED_DIM):
                col = jnp.where(lane_iota >= EMBED_DIM - d,
                                lane_iota + (d - EMBED_DIM),
                                lane_iota + d)
                w_d = plsc.load_gather(wrows_v, [bvec, col])
                for g in range(4):
                    v = plsc.load_gather(crows_v, [rows[g], col])
                    accs[g] = accs[g] + v * w_d
            for g in range(4):
                sig = 1.0 / (1.0 + jnp.exp(-accs[g]))
                cvec = g * LANES + lane_iota
                if g * LANES + LANES <= CTX:
                    plsc.store_scatter(out_v, [cvec, bw], sig)
                else:
                    plsc.store_scatter(out_v, [cvec, bw], sig,
                                       mask=lane_iota < (CTX - g * LANES))
            return ()

        lax.fori_loop(0, CB, batch_body, (), unroll=False)
        return ()

    lax.fori_loop(0, N_CHUNKS, chunk_body, (), unroll=False)
    pltpu.sync_copy(out_v.at[:, pl.ds(0, B_PER_W)],
                    out_hbm.at[:, pl.ds(row0, B_PER_W)])


@functools.partial(
    pl.kernel,
    out_type=jax.ShapeDtypeStruct((CTX, BATCH), jnp.float32),
    mesh=plsc.VectorSubcoreMesh(core_axis_name="c", subcore_axis_name="s"),
    compiler_params=pltpu.CompilerParams(
        needs_layout_passes=False, use_tc_tiling_on_sc=False),
    scratch_types=[
        pltpu.VMEM((XW, B_PER_W), jnp.int32),
        pltpu.VMEM((B_PER_W,), jnp.int32),
        pltpu.VMEM((ROWS_PER_W,), jnp.int32),
        pltpu.VMEM((2 * CB, EMBED_DIM), jnp.float32),
        pltpu.VMEM((2 * CPAD, EMBED_DIM), jnp.float32),
        pltpu.VMEM((CTX, OW), jnp.float32),
        pltpu.SemaphoreType.DMA,
    ],
)
def _w2v_sc(x_hbm, wtab_hbm, ctab_hbm, out_hbm,
            xs_v, widx_v, cidx_v, wrows_v, crows_v, out_v, gsem):
    _body(x_hbm, wtab_hbm, ctab_hbm, out_hbm,
          xs_v, widx_v, cidx_v, wrows_v, crows_v, out_v, gsem)


def kernel(x, word_emb, ctx_emb):
    out = _w2v_sc(x.T, word_emb, ctx_emb)
    return out.T[:, :, None]


# speedup vs baseline: 1.0258x; 1.0240x over previous
import functools

import jax
import jax.numpy as jnp
from jax import lax
from jax.experimental import pallas as pl
from jax.experimental.pallas import tpu as pltpu, tpu_sc as plsc

VOCAB = 100000
EMBED_DIM = 64
BATCH = 4096
CTX = 50
XW = CTX + 1
XP = 128

NC, NS, LANES = 2, 16, 16
NW = NC * NS
B_PER_W = BATCH // NW
CB = 8
N_CHUNKS = B_PER_W // CB
ROWS_PER_CHUNK = CB * CTX
ROWS_PER_W = B_PER_W * CTX
IDX_CHUNK = 128
DIV50_MUL, DIV50_SHIFT = 20972, 20

CPAD = ROWS_PER_CHUNK + LANES
OW = B_PER_W + 1


def _body(x_hbm, wtab_hbm, ctab_hbm, out_hbm,
          xs_v, widx_v, cidx_v, wrows_v, crows_v, out_v, gsem):
    wid = lax.axis_index("s") * NC + lax.axis_index("c")
    lane_iota = lax.iota(jnp.int32, LANES)
    row0 = wid * B_PER_W

    pltpu.sync_copy(x_hbm.at[pl.ds(row0, B_PER_W)], xs_v)
    zvec = jnp.zeros((LANES,), jnp.int32)
    for k in range(B_PER_W // LANES):
        widx_v[pl.ds(k * LANES, LANES)] = plsc.load_gather(
            xs_v, [k * LANES + lane_iota, zvec])

    def extract_body(g2, _):
        o = g2 * LANES + lane_iota
        j = (o * DIV50_MUL) >> DIV50_SHIFT
        p1 = o - j * CTX + 1
        cidx_v[pl.ds(g2 * LANES, LANES)] = plsc.load_gather(xs_v, [j, p1])
        return ()

    lax.fori_loop(0, ROWS_PER_W // LANES, extract_body, (), unroll=False)

    def fire_gathers(k):
        p = lax.rem(k, 2)
        pltpu.async_copy(
            wtab_hbm.at[widx_v.at[pl.ds(k * CB, CB)]],
            wrows_v.at[pl.ds(p * CB, CB)], gsem)
        off = 0
        while off < ROWS_PER_CHUNK:
            n = min(IDX_CHUNK, ROWS_PER_CHUNK - off)
            pltpu.async_copy(
                ctab_hbm.at[cidx_v.at[pl.ds(k * ROWS_PER_CHUNK + off, n)]],
                crows_v.at[pl.ds(p * CPAD + off, n)], gsem)
            off += n

    def drain_gathers():
        pltpu.make_async_copy(ctab_hbm.at[pl.ds(0, ROWS_PER_CHUNK)],
                              crows_v.at[pl.ds(0, ROWS_PER_CHUNK)],
                              gsem).wait()
        pltpu.make_async_copy(wtab_hbm.at[pl.ds(0, CB)],
                              wrows_v.at[pl.ds(0, CB)], gsem).wait()

    fire_gathers(jnp.int32(0))

    def chunk_body(k, _):
        p = lax.rem(k, 2)

        @pl.when(k + 1 < N_CHUNKS)
        def _():
            fire_gathers(k + 1)

        drain_gathers()

        poff_c = p * CPAD
        poff_w = p * CB

        def batch_body(b, _):
            bvec = jnp.broadcast_to(poff_w + b, (LANES,)).astype(jnp.int32)
            bw = jnp.broadcast_to(k * CB + b, (LANES,)).astype(jnp.int32)
            rows = [poff_c + b * CTX + g * LANES + lane_iota for g in range(4)]
            accs = [jnp.zeros((LANES,), jnp.float32) for _ in range(4)]
            for d in range(EMBED_DIM):
                col = jnp.where(lane_iota >= EMBED_DIM - d,
                                lane_iota + (d - EMBED_DIM),
                                lane_iota + d)
                w_d = plsc.load_gather(wrows_v, [bvec, col])
                for g in range(4):
                    v = plsc.load_gather(crows_v, [rows[g], col])
                    accs[g] = accs[g] + v * w_d
            for g in range(4):
                sig = 1.0 / (1.0 + jnp.exp(-accs[g]))
                cvec = g * LANES + lane_iota
                if g * LANES + LANES <= CTX:
                    plsc.store_scatter(out_v, [cvec, bw], sig)
                else:
                    plsc.store_scatter(out_v, [cvec, bw], sig,
                                       mask=lane_iota < (CTX - g * LANES))
            return ()

        lax.fori_loop(0, CB, batch_body, (), unroll=False)
        return ()

    lax.fori_loop(0, N_CHUNKS, chunk_body, (), unroll=False)
    pltpu.sync_copy(out_v.at[:, pl.ds(0, B_PER_W)],
                    out_hbm.at[:, pl.ds(row0, B_PER_W)])


@functools.partial(
    pl.kernel,
    out_type=jax.ShapeDtypeStruct((CTX, BATCH), jnp.float32),
    mesh=plsc.VectorSubcoreMesh(core_axis_name="c", subcore_axis_name="s"),
    compiler_params=pltpu.CompilerParams(
        needs_layout_passes=False, use_tc_tiling_on_sc=False),
    scratch_types=[
        pltpu.VMEM((B_PER_W, XP), jnp.int32),
        pltpu.VMEM((B_PER_W,), jnp.int32),
        pltpu.VMEM((ROWS_PER_W,), jnp.int32),
        pltpu.VMEM((2 * CB, EMBED_DIM), jnp.float32),
        pltpu.VMEM((2 * CPAD, EMBED_DIM), jnp.float32),
        pltpu.VMEM((CTX, OW), jnp.float32),
        pltpu.SemaphoreType.DMA,
    ],
)
def _w2v_sc(x_hbm, wtab_hbm, ctab_hbm, out_hbm,
            xs_v, widx_v, cidx_v, wrows_v, crows_v, out_v, gsem):
    _body(x_hbm, wtab_hbm, ctab_hbm, out_hbm,
          xs_v, widx_v, cidx_v, wrows_v, crows_v, out_v, gsem)


def kernel(x, word_emb, ctx_emb):
    xp = jnp.pad(x, ((0, 0), (0, XP - XW)))
    out = _w2v_sc(xp, word_emb, ctx_emb)
    return out.T[:, :, None]
